# baseline (speedup 1.0000x reference)
import functools
import math

import jax
import jax.numpy as jnp
from jax.experimental import pallas as pl
from jax.experimental.pallas import tpu as pltpu


def _scatter_body(idx_ref, value_ref, store_ref, out_ref):
    c = pl.program_id(0)
    R = out_ref.shape[0]
    B = value_ref.shape[0]
    rows = jax.lax.broadcasted_iota(jnp.int32, (R, B), 0) + c * R
    biota = jax.lax.broadcasted_iota(jnp.int32, (R, B), 1)
    idx = idx_ref[0, :]
    hit = rows == idx[None, :]
    wb = jnp.max(jnp.where(hit, biota, -1), axis=1, keepdims=True)
    P = (biota == wb).astype(jnp.float32)
    corr = jax.lax.dot_general(
        P, value_ref[...], (((1,), (0,)), ((), ())),
        preferred_element_type=jnp.float32)
    out_ref[...] = jnp.where(wb >= 0, corr, store_ref[...])


def _sem_body(q_ref, k_ref, v_ref, out_ref, m_s, l_s, acc_s, *, scale, nk):
    j = pl.program_id(1)

    @pl.when(j == 0)
    def _():
        m_s[...] = jnp.full_like(m_s[...], -jnp.inf)
        l_s[...] = jnp.zeros_like(l_s[...])
        acc_s[...] = jnp.zeros_like(acc_s[...])

    qb = q_ref[...].astype(jnp.bfloat16)
    kb = k_ref[...].astype(jnp.bfloat16)
    s = jax.lax.dot_general(
        qb, kb, (((1,), (1,)), ((), ())),
        preferred_element_type=jnp.float32) * scale
    m_old = m_s[...]
    m_new = jnp.maximum(m_old, jnp.max(s, axis=1, keepdims=True))
    alpha = jnp.exp(m_old - m_new)
    p = jnp.exp(s - m_new[:, :1])
    l_s[...] = l_s[...] * alpha + jnp.sum(p, axis=1, keepdims=True)
    m_s[...] = m_new
    pv = jax.lax.dot_general(
        p.astype(jnp.bfloat16), v_ref[...].astype(jnp.bfloat16),
        (((1,), (0,)), ((), ())), preferred_element_type=jnp.float32)
    acc_s[...] = acc_s[...] * alpha[:, :1] + pv

    @pl.when(j == nk - 1)
    def _():
        out_ref[...] = 0.425 * acc_s[...] / l_s[...][:, :1]


def _ep_body(q_ref, ep_ref, partial_ref, out_ref,
             m1, l1, acc1, m2, l2, acc2, *, scale, beta, nk):
    j = pl.program_id(1)

    @pl.when(j == 0)
    def _():
        for m_s, l_s, acc_s in ((m1, l1, acc1), (m2, l2, acc2)):
            m_s[...] = jnp.full_like(m_s[...], -jnp.inf)
            l_s[...] = jnp.zeros_like(l_s[...])
            acc_s[...] = jnp.zeros_like(acc_s[...])

    ep = ep_ref[...]
    s0 = jax.lax.dot_general(
        q_ref[...], ep, (((1,), (1,)), ((), ())),
        preferred_element_type=jnp.float32)
    for m_s, l_s, acc_s, t in ((m1, l1, acc1, scale), (m2, l2, acc2, beta)):
        s = s0 * t
        m_old = m_s[...]
        m_new = jnp.maximum(m_old, jnp.max(s, axis=1, keepdims=True))
        alpha = jnp.exp(m_old - m_new)
        p = jnp.exp(s - m_new[:, :1])
        l_s[...] = l_s[...] * alpha + jnp.sum(p, axis=1, keepdims=True)
        m_s[...] = m_new
        pv = jax.lax.dot_general(
            p, ep, (((1,), (0,)), ((), ())),
            preferred_element_type=jnp.float32)
        acc_s[...] = acc_s[...] * alpha[:, :1] + pv

    @pl.when(j == nk - 1)
    def _():
        out_ref[...] = (partial_ref[...]
                        + 0.425 * acc1[...] / l1[...][:, :1]
                        + 0.15 * acc2[...] / l2[...][:, :1])


def kernel(query, value, episodic_store, semantic_keys, semantic_values,
           write_idx):
    B, D = query.shape
    EP = episodic_store.shape[0]
    SEM = semantic_keys.shape[0]
    scale = 1.0 / math.sqrt(D)
    beta = 2.0

    BQ = 256
    BK_SEM = 2048
    BK_EP = 2048
    nq = B // BQ
    nk_sem = SEM // BK_SEM
    nk_ep = EP // BK_EP

    idx2d = write_idx.astype(jnp.int32).reshape(1, B)

    partial = pl.pallas_call(
        functools.partial(_sem_body, scale=scale, nk=nk_sem),
        grid=(nq, nk_sem),
        in_specs=[
            pl.BlockSpec((BQ, D), lambda i, j: (i, 0)),
            pl.BlockSpec((BK_SEM, D), lambda i, j: (j, 0)),
            pl.BlockSpec((BK_SEM, D), lambda i, j: (j, 0)),
        ],
        out_specs=pl.BlockSpec((BQ, D), lambda i, j: (i, 0)),
        out_shape=jax.ShapeDtypeStruct((B, D), jnp.float32),
        scratch_shapes=[
            pltpu.VMEM((BQ, 128), jnp.float32),
            pltpu.VMEM((BQ, 128), jnp.float32),
            pltpu.VMEM((BQ, D), jnp.float32),
        ],
        compiler_params=pltpu.CompilerParams(
            dimension_semantics=("arbitrary", "arbitrary")),
    )(query, semantic_keys, semantic_values)

    RCH = 512
    ep = pl.pallas_call(
        _scatter_body,
        grid=(EP // RCH,),
        in_specs=[
            pl.BlockSpec((1, B), lambda c: (0, 0)),
            pl.BlockSpec((B, D), lambda c: (0, 0)),
            pl.BlockSpec((RCH, D), lambda c: (c, 0)),
        ],
        out_specs=pl.BlockSpec((RCH, D), lambda c: (c, 0)),
        out_shape=jax.ShapeDtypeStruct((EP, D), jnp.float32),
    )(idx2d, value, episodic_store)

    out = pl.pallas_call(
        functools.partial(_ep_body, scale=scale, beta=beta, nk=nk_ep),
        grid=(nq, nk_ep),
        in_specs=[
            pl.BlockSpec((BQ, D), lambda i, j: (i, 0)),
            pl.BlockSpec((BK_EP, D), lambda i, j: (j, 0)),
            pl.BlockSpec((BQ, D), lambda i, j: (i, 0)),
        ],
        out_specs=pl.BlockSpec((BQ, D), lambda i, j: (i, 0)),
        out_shape=jax.ShapeDtypeStruct((B, D), jnp.float32),
        scratch_shapes=[
            pltpu.VMEM((BQ, 128), jnp.float32),
            pltpu.VMEM((BQ, 128), jnp.float32),
            pltpu.VMEM((BQ, D), jnp.float32),
            pltpu.VMEM((BQ, 128), jnp.float32),
            pltpu.VMEM((BQ, 128), jnp.float32),
            pltpu.VMEM((BQ, D), jnp.float32),
        ],
        compiler_params=pltpu.CompilerParams(
            dimension_semantics=("arbitrary", "arbitrary")),
    )(query, ep, partial)

    return out


# speedup vs baseline: 1.2350x; 1.2350x over previous
import functools
import math

import jax
import jax.numpy as jnp
from jax.experimental import pallas as pl
from jax.experimental.pallas import tpu as pltpu


def _scatter_body(idx_ref, value_ref, store_ref, out_ref):
    c = pl.program_id(0)
    R = out_ref.shape[0]
    B = value_ref.shape[0]
    rows = jax.lax.broadcasted_iota(jnp.int32, (R, B), 0) + c * R
    biota = jax.lax.broadcasted_iota(jnp.int32, (R, B), 1)
    idx = idx_ref[0, :]
    hit = rows == idx[None, :]
    wb = jnp.max(jnp.where(hit, biota, -1), axis=1, keepdims=True)
    P = (biota == wb).astype(jnp.float32)
    corr = jax.lax.dot_general(
        P, value_ref[...], (((1,), (0,)), ((), ())),
        preferred_element_type=jnp.float32)
    out_ref[...] = jnp.where(wb >= 0, corr, store_ref[...])


def _sem_body(q_ref, k_ref, v_ref, out_ref, m_s, l_s, acc_s, *, scale, nk):
    j = pl.program_id(0)

    @pl.when(j == 0)
    def _():
        m_s[...] = jnp.full_like(m_s[...], -jnp.inf)
        l_s[...] = jnp.zeros_like(l_s[...])
        acc_s[...] = jnp.zeros_like(acc_s[...])

    qb = q_ref[...].astype(jnp.bfloat16)
    kb = k_ref[...].astype(jnp.bfloat16)
    s = jax.lax.dot_general(
        qb, kb, (((1,), (1,)), ((), ())),
        preferred_element_type=jnp.float32) * scale
    m_old = m_s[...]
    m_new = jnp.maximum(m_old, jnp.max(s, axis=1, keepdims=True))
    alpha = jnp.exp(m_old - m_new)
    p = jnp.exp(s - m_new[:, :1])
    l_s[...] = l_s[...] * alpha + jnp.sum(p, axis=1, keepdims=True)
    m_s[...] = m_new
    pv = jax.lax.dot_general(
        p.astype(jnp.bfloat16), v_ref[...].astype(jnp.bfloat16),
        (((1,), (0,)), ((), ())), preferred_element_type=jnp.float32)
    acc_s[...] = acc_s[...] * alpha[:, :1] + pv

    @pl.when(j == nk - 1)
    def _():
        out_ref[...] = 0.425 * acc_s[...] / l_s[...][:, :1]


def _ep_body(q_ref, ep_ref, partial_ref, out_ref,
             m1, l1, acc1, m2, l2, acc2, *, scale, beta, nk):
    j = pl.program_id(0)

    @pl.when(j == 0)
    def _():
        for m_s, l_s, acc_s in ((m1, l1, acc1), (m2, l2, acc2)):
            m_s[...] = jnp.full_like(m_s[...], -jnp.inf)
            l_s[...] = jnp.zeros_like(l_s[...])
            acc_s[...] = jnp.zeros_like(acc_s[...])

    ep = ep_ref[...]
    s0 = jax.lax.dot_general(
        q_ref[...], ep, (((1,), (1,)), ((), ())),
        preferred_element_type=jnp.float32)
    for m_s, l_s, acc_s, t in ((m1, l1, acc1, scale), (m2, l2, acc2, beta)):
        s = s0 * t
        m_old = m_s[...]
        m_new = jnp.maximum(m_old, jnp.max(s, axis=1, keepdims=True))
        alpha = jnp.exp(m_old - m_new)
        p = jnp.exp(s - m_new[:, :1])
        l_s[...] = l_s[...] * alpha + jnp.sum(p, axis=1, keepdims=True)
        m_s[...] = m_new
        pv = jax.lax.dot_general(
            p.astype(jnp.bfloat16), ep.astype(jnp.bfloat16),
            (((1,), (0,)), ((), ())),
            preferred_element_type=jnp.float32)
        acc_s[...] = acc_s[...] * alpha[:, :1] + pv

    @pl.when(j == nk - 1)
    def _():
        out_ref[...] = (partial_ref[...]
                        + 0.425 * acc1[...] / l1[...][:, :1]
                        + 0.15 * acc2[...] / l2[...][:, :1])


def kernel(query, value, episodic_store, semantic_keys, semantic_values,
           write_idx):
    B, D = query.shape
    EP = episodic_store.shape[0]
    SEM = semantic_keys.shape[0]
    scale = 1.0 / math.sqrt(D)
    beta = 2.0

    BQ = 1024
    BK_SEM = 1024
    BK_EP = 1024
    nk_sem = SEM // BK_SEM
    nk_ep = EP // BK_EP

    idx2d = write_idx.astype(jnp.int32).reshape(1, B)

    partial = pl.pallas_call(
        functools.partial(_sem_body, scale=scale, nk=nk_sem),
        grid=(nk_sem,),
        in_specs=[
            pl.BlockSpec((BQ, D), lambda j: (0, 0)),
            pl.BlockSpec((BK_SEM, D), lambda j: (j, 0)),
            pl.BlockSpec((BK_SEM, D), lambda j: (j, 0)),
        ],
        out_specs=pl.BlockSpec((BQ, D), lambda j: (0, 0)),
        out_shape=jax.ShapeDtypeStruct((B, D), jnp.float32),
        scratch_shapes=[
            pltpu.VMEM((BQ, 128), jnp.float32),
            pltpu.VMEM((BQ, 128), jnp.float32),
            pltpu.VMEM((BQ, D), jnp.float32),
        ],
        compiler_params=pltpu.CompilerParams(
            dimension_semantics=("arbitrary",)),
    )(query, semantic_keys, semantic_values)

    RCH = 512
    ep = pl.pallas_call(
        _scatter_body,
        grid=(EP // RCH,),
        in_specs=[
            pl.BlockSpec((1, B), lambda c: (0, 0)),
            pl.BlockSpec((B, D), lambda c: (0, 0)),
            pl.BlockSpec((RCH, D), lambda c: (c, 0)),
        ],
        out_specs=pl.BlockSpec((RCH, D), lambda c: (c, 0)),
        out_shape=jax.ShapeDtypeStruct((EP, D), jnp.float32),
    )(idx2d, value, episodic_store)

    out = pl.pallas_call(
        functools.partial(_ep_body, scale=scale, beta=beta, nk=nk_ep),
        grid=(nk_ep,),
        in_specs=[
            pl.BlockSpec((BQ, D), lambda j: (0, 0)),
            pl.BlockSpec((BK_EP, D), lambda j: (j, 0)),
            pl.BlockSpec((BQ, D), lambda j: (0, 0)),
        ],
        out_specs=pl.BlockSpec((BQ, D), lambda j: (0, 0)),
        out_shape=jax.ShapeDtypeStruct((B, D), jnp.float32),
        scratch_shapes=[
            pltpu.VMEM((BQ, 128), jnp.float32),
            pltpu.VMEM((BQ, 128), jnp.float32),
            pltpu.VMEM((BQ, D), jnp.float32),
            pltpu.VMEM((BQ, 128), jnp.float32),
            pltpu.VMEM((BQ, 128), jnp.float32),
            pltpu.VMEM((BQ, D), jnp.float32),
        ],
        compiler_params=pltpu.CompilerParams(
            dimension_semantics=("arbitrary",)),
    )(query, ep, partial)

    return out
